# baseline (speedup 1.0000x reference)
import jax
import jax.numpy as jnp
import numpy as np
from jax import lax
from jax.experimental import pallas as pl
from jax.experimental.pallas import tpu as pltpu

K = 4
HM = 64
OUT = 5
C = 128

_P = np.zeros((25, 14), np.float32)
for _i in range(14):
    _a = (_i * 25) // 14
    _b = int(np.ceil((_i + 1) * 25 / 14))
    _P[_a:_b, _i] = 1.0 / (_b - _a)

_HI = lax.Precision.HIGHEST


def _ax(lo, d, offs):
    s = lo + offs * d
    v = (s > -1.0) & (s < float(HM))
    sc = jnp.clip(s, 0.0, float(HM - 1))
    fl = jnp.floor(sc)
    fr = sc - fl
    fh = jnp.minimum(fl + 1.0, float(HM - 1))
    return v, fl, fh, fr


def _body(hm_ref, off_ref, bsz_ref, hft_ref, vft_ref, ewt_ref, eb_ref,
          pw_ref, pbc_ref, pool_ref, o_ref):
    f32 = jnp.float32
    dn = (((1,), (1,)), ((), ()))
    hm = hm_ref[0, 0]
    ri = lax.broadcasted_iota(jnp.int32, (HM, HM), 0)
    ci = lax.broadcasted_iota(jnp.int32, (HM, HM), 1)
    lin = ri * HM + ci
    pos64 = lax.broadcasted_iota(jnp.int32, (OUT, HM), 1).astype(f32)
    offs_c = [lax.broadcasted_iota(jnp.int32, (OUT, 1), 0).astype(f32) + 0.25,
              lax.broadcasted_iota(jnp.int32, (OUT, 1), 0).astype(f32) + 0.75]
    offs_x3 = [lax.broadcasted_iota(jnp.int32, (1, OUT, 1), 1).astype(f32) + 0.25,
               lax.broadcasted_iota(jnp.int32, (1, OUT, 1), 1).astype(f32) + 0.75]
    offs_y3 = [lax.broadcasted_iota(jnp.int32, (OUT, 1, 1), 0).astype(f32) + 0.25,
               lax.broadcasted_iota(jnp.int32, (OUT, 1, 1), 0).astype(f32) + 0.75]

    def onehot(coord):
        return (pos64 == coord).astype(f32)

    rhv5 = jnp.float32(64.0 / OUT)
    gyv_blocks = []
    lyv3, vyv3 = [], []
    for par in range(2):
        _, flv, fhv, _ = _ax(0.0, rhv5, offs_c[par])
        gyv_blocks.append((onehot(flv), onehot(fhv)))
        vv3, _, _, frv3 = _ax(0.0, rhv5, offs_y3[par])
        lyv3.append(frv3)
        vyv3.append(vv3)
    gyv = jnp.concatenate([gyv_blocks[0][0], gyv_blocks[1][0],
                           gyv_blocks[0][1], gyv_blocks[1][1]], axis=0)

    ewt = ewt_ref[...]
    eb = eb_ref[...]
    pw = pw_ref[...]
    pbc = pbc_ref[...]
    pool = pool_ref[...]

    cur = hm
    rois = []
    for _ in range(K):
        mx = jnp.max(cur, axis=(0, 1), keepdims=True)
        idx = jnp.min(jnp.where(cur == mx, lin, HM * HM),
                      axis=(0, 1), keepdims=True)
        sel = lin == idx
        cur = jnp.where(sel, -jnp.inf, cur)
        self_ = sel.astype(f32)
        y = (idx // HM).astype(f32)
        x = (idx % HM).astype(f32)
        o0 = jnp.sum(off_ref[0, 0] * self_, axis=(0, 1), keepdims=True)
        o1 = jnp.sum(off_ref[0, 1] * self_, axis=(0, 1), keepdims=True)
        b0 = jnp.sum(bsz_ref[0, 0] * self_, axis=(0, 1), keepdims=True)
        b1 = jnp.sum(bsz_ref[0, 1] * self_, axis=(0, 1), keepdims=True)
        b2 = jnp.sum(bsz_ref[0, 2] * self_, axis=(0, 1), keepdims=True)
        b3 = jnp.sum(bsz_ref[0, 3] * self_, axis=(0, 1), keepdims=True)
        wdt = b0 + b2
        hgt = b1 + b3
        cx = (x + o0) * 4.0
        cy = (y + o1) * 4.0
        x1 = (cx - wdt / 2) * 0.25
        y1 = (cy - hgt / 2) * 0.25
        x2 = (cx + wdt / 2) * 0.25
        y2 = (cy + hgt / 2) * 0.25
        rw5 = jnp.maximum(x2 - x1, 1.0) / OUT
        rh5 = jnp.maximum(y2 - y1, 1.0) / OUT
        gx_blocks, gy_blocks = [], []
        lx3, vx3, ly3, vy3 = [], [], [], []
        for par in range(2):
            _, flx, fhx, _ = _ax(x1, rw5, offs_c[par])
            gx_blocks.append((onehot(flx), onehot(fhx)))
            _, fly, fhy, _ = _ax(y1, rh5, offs_c[par])
            gy_blocks.append((onehot(fly), onehot(fhy)))
            vx, _, _, frx = _ax(x1, rw5, offs_x3[par])
            lx3.append(frx)
            vx3.append(vx)
            vy, _, _, fry = _ax(y1, rh5, offs_y3[par])
            ly3.append(fry)
            vy3.append(vy)
        gx = jnp.concatenate([gx_blocks[0][0], gx_blocks[1][0],
                              gx_blocks[0][1], gx_blocks[1][1]], axis=0)
        gy = jnp.concatenate([gy_blocks[0][0], gy_blocks[1][0],
                              gy_blocks[0][1], gy_blocks[1][1]], axis=0)
        rois.append(dict(x1=x1, y1=y1, x2=x2, y2=y2, gx=gx, gy=gy,
                         lx3=lx3, vx3=vx3, ly3=ly3, vy3=vy3))

    dn2 = (((1,), (2,)), ((), ()))
    gx_all = jnp.concatenate([r["gx"] for r in rois], axis=0)
    th = lax.dot_general(gx_all, hft_ref[0], dn2, precision=_HI,
                         preferred_element_type=f32)
    tv = lax.dot_general(gx_all, vft_ref[0], dn2, precision=_HI,
                         preferred_element_type=f32)

    def bilinear_mean(u, lx3, vx3, ly3, vy3):
        quad = [[None, None], [None, None]]
        for p in range(2):
            for q in range(2):
                v00 = u[5 * q:5 * q + 5, 5 * p:5 * p + 5, :]
                v01 = u[5 * q:5 * q + 5, 5 * (2 + p):5 * (2 + p) + 5, :]
                v10 = u[5 * (2 + q):5 * (2 + q) + 5, 5 * p:5 * p + 5, :]
                v11 = u[5 * (2 + q):5 * (2 + q) + 5,
                        5 * (2 + p):5 * (2 + p) + 5, :]
                o = (v00 * ((1 - ly3[q]) * (1 - lx3[p]))
                     + v01 * ((1 - ly3[q]) * lx3[p])
                     + v10 * (ly3[q] * (1 - lx3[p]))
                     + v11 * (ly3[q] * lx3[p]))
                quad[p][q] = jnp.where(vy3[q] & vx3[p], o, 0.0)
        return ((quad[0][0] + quad[0][1]) + (quad[1][0] + quad[1][1])) / 4

    for r in range(K):
        roi = rois[r]
        th_r = th[20 * r:20 * r + 20]
        tv_r = tv[20 * r:20 * r + 20]
        uh = lax.dot_general(roi["gy"], th_r, dn2, precision=_HI,
                             preferred_element_type=f32)
        uv = lax.dot_general(gyv, tv_r, dn2, precision=_HI,
                             preferred_element_type=f32)
        mh = bilinear_mean(uh, roi["lx3"], roi["vx3"], roi["ly3"], roi["vy3"])
        mv = bilinear_mean(uv, roi["lx3"], roi["vx3"], lyv3, vyv3)
        afh = lax.dot_general(mh.reshape(25, C), pool, (((0,), (0,)), ((), ())),
                              preferred_element_type=f32)
        afv = lax.dot_general(mv.reshape(25, C), pool, (((0,), (0,)), ((), ())),
                              preferred_element_type=f32)
        x1, y1, x2, y2 = roi["x1"], roi["y1"], roi["x2"], roi["y2"]
        v4h = jnp.concatenate([x1 / 64.0, y1 / 64.0, x2 / 64.0, y2 / 64.0],
                              axis=1)
        zero = x1 * 0.0
        v4v = jnp.concatenate([x1 / 64.0, zero, x2 / 64.0, zero + 1.0], axis=1)
        embh = jnp.transpose(
            lax.dot_general(v4h, ewt, (((1,), (0,)), ((), ())),
                            preferred_element_type=f32) + eb)
        embv = jnp.transpose(
            lax.dot_general(v4v, ewt, (((1,), (0,)), ((), ())),
                            preferred_element_type=f32) + eb)
        afh = afh + embh
        afv = afv + embv
        gh = lax.dot_general(pw, afh, (((1,), (0,)), ((), ())),
                             preferred_element_type=f32) + pbc
        gv = lax.dot_general(pw, afv, (((1,), (0,)), ((), ())),
                             preferred_element_type=f32) + pbc
        s = gh + gv
        wh = jnp.exp(gh) / s
        wv = jnp.exp(gv) / s
        o_ref[0, r] = gh * wh + gv * wv


@jax.jit
def kernel(hor_heatmap, hor_offset, hor_bsize, hor_kpt_feats, ver_kpt_feats,
           embed_w, embed_b, proj_w, proj_b):
    n = hor_heatmap.shape[0]
    ewt = jnp.transpose(embed_w)
    eb = embed_b.reshape(1, C)
    pbc = proj_b.reshape(C, 1)
    pool = jnp.asarray(_P)
    return pl.pallas_call(
        _body,
        grid=(n,),
        in_specs=[
            pl.BlockSpec((1, 1, HM, HM), lambda i: (i, 0, 0, 0)),
            pl.BlockSpec((1, 2, HM, HM), lambda i: (i, 0, 0, 0)),
            pl.BlockSpec((1, 4, HM, HM), lambda i: (i, 0, 0, 0)),
            pl.BlockSpec((1, C, HM, HM), lambda i: (i, 0, 0, 0)),
            pl.BlockSpec((1, C, HM, HM), lambda i: (i, 0, 0, 0)),
            pl.BlockSpec((4, C), lambda i: (0, 0)),
            pl.BlockSpec((1, C), lambda i: (0, 0)),
            pl.BlockSpec((C, C), lambda i: (0, 0)),
            pl.BlockSpec((C, 1), lambda i: (0, 0)),
            pl.BlockSpec((25, 14), lambda i: (0, 0)),
        ],
        out_specs=pl.BlockSpec((1, K, C, 14), lambda i: (i, 0, 0, 0)),
        out_shape=jax.ShapeDtypeStruct((n, K, C, 14), jnp.float32),
        compiler_params=pltpu.CompilerParams(
            dimension_semantics=("parallel",),
            vmem_limit_bytes=48 * 1024 * 1024,
        ),
    )(hor_heatmap, hor_offset, hor_bsize, hor_kpt_feats, ver_kpt_feats,
      ewt, eb, proj_w, pbc, pool)


# speedup vs baseline: 2.4088x; 2.4088x over previous
import jax
import jax.numpy as jnp
import numpy as np
from jax import lax
from jax.experimental import pallas as pl
from jax.experimental.pallas import tpu as pltpu

K = 4
HM = 64
OUT = 5
C = 128

_P = np.zeros((25, 14), np.float32)
for _i in range(14):
    _a = (_i * 25) // 14
    _b = int(np.ceil((_i + 1) * 25 / 14))
    _P[_a:_b, _i] = 1.0 / (_b - _a)

_HI = lax.Precision.HIGHEST


def _ax(lo, d, offs):
    s = lo + offs * d
    v = (s > -1.0) & (s < float(HM))
    sc = jnp.clip(s, 0.0, float(HM - 1))
    fl = jnp.floor(sc)
    fr = sc - fl
    fh = jnp.minimum(fl + 1.0, float(HM - 1))
    return v, fl, fh, fr


def _body(hm_ref, off_ref, bsz_ref, hft_ref, vft_ref, ewt_ref, eb_ref,
          pw_ref, pbc_ref, pool_ref, o_ref):
    f32 = jnp.float32
    dn = (((1,), (1,)), ((), ()))
    hm = hm_ref[0, 0]
    ri = lax.broadcasted_iota(jnp.int32, (HM, HM), 0)
    ci = lax.broadcasted_iota(jnp.int32, (HM, HM), 1)
    lin = ri * HM + ci
    pos64 = lax.broadcasted_iota(jnp.int32, (OUT, HM), 1).astype(f32)
    offs_c = [lax.broadcasted_iota(jnp.int32, (OUT, 1), 0).astype(f32) + 0.25,
              lax.broadcasted_iota(jnp.int32, (OUT, 1), 0).astype(f32) + 0.75]
    offs_x3 = [lax.broadcasted_iota(jnp.int32, (1, OUT, 1), 1).astype(f32) + 0.25,
               lax.broadcasted_iota(jnp.int32, (1, OUT, 1), 1).astype(f32) + 0.75]
    offs_y3 = [lax.broadcasted_iota(jnp.int32, (OUT, 1, 1), 0).astype(f32) + 0.25,
               lax.broadcasted_iota(jnp.int32, (OUT, 1, 1), 0).astype(f32) + 0.75]

    def onehot(coord):
        return (pos64 == coord).astype(f32)

    rhv5 = jnp.float32(64.0 / OUT)
    gyv_blocks = []
    lyv3, vyv3 = [], []
    for par in range(2):
        _, flv, fhv, _ = _ax(0.0, rhv5, offs_c[par])
        gyv_blocks.append((onehot(flv), onehot(fhv)))
        vv3, _, _, frv3 = _ax(0.0, rhv5, offs_y3[par])
        lyv3.append(frv3)
        vyv3.append(vv3)
    gyv = jnp.concatenate([gyv_blocks[0][0], gyv_blocks[1][0],
                           gyv_blocks[0][1], gyv_blocks[1][1]], axis=0)

    ewt = ewt_ref[...]
    eb = eb_ref[...]
    pw = pw_ref[...]
    pbc = pbc_ref[...]
    pool = pool_ref[...]

    cur = hm
    rois = []
    for _ in range(K):
        mx = jnp.max(cur, axis=(0, 1), keepdims=True)
        idx = jnp.min(jnp.where(cur == mx, lin, HM * HM),
                      axis=(0, 1), keepdims=True)
        sel = lin == idx
        cur = jnp.where(sel, -jnp.inf, cur)
        self_ = sel.astype(f32)
        y = (idx // HM).astype(f32)
        x = (idx % HM).astype(f32)
        o0 = jnp.sum(off_ref[0, 0] * self_, axis=(0, 1), keepdims=True)
        o1 = jnp.sum(off_ref[0, 1] * self_, axis=(0, 1), keepdims=True)
        b0 = jnp.sum(bsz_ref[0, 0] * self_, axis=(0, 1), keepdims=True)
        b1 = jnp.sum(bsz_ref[0, 1] * self_, axis=(0, 1), keepdims=True)
        b2 = jnp.sum(bsz_ref[0, 2] * self_, axis=(0, 1), keepdims=True)
        b3 = jnp.sum(bsz_ref[0, 3] * self_, axis=(0, 1), keepdims=True)
        wdt = b0 + b2
        hgt = b1 + b3
        cx = (x + o0) * 4.0
        cy = (y + o1) * 4.0
        x1 = (cx - wdt / 2) * 0.25
        y1 = (cy - hgt / 2) * 0.25
        x2 = (cx + wdt / 2) * 0.25
        y2 = (cy + hgt / 2) * 0.25
        rw5 = jnp.maximum(x2 - x1, 1.0) / OUT
        rh5 = jnp.maximum(y2 - y1, 1.0) / OUT
        gx_blocks, gy_blocks = [], []
        lx3, vx3, ly3, vy3 = [], [], [], []
        for par in range(2):
            _, flx, fhx, _ = _ax(x1, rw5, offs_c[par])
            gx_blocks.append((onehot(flx), onehot(fhx)))
            _, fly, fhy, _ = _ax(y1, rh5, offs_c[par])
            gy_blocks.append((onehot(fly), onehot(fhy)))
            vx, _, _, frx = _ax(x1, rw5, offs_x3[par])
            lx3.append(frx)
            vx3.append(vx)
            vy, _, _, fry = _ax(y1, rh5, offs_y3[par])
            ly3.append(fry)
            vy3.append(vy)
        gx = jnp.concatenate([gx_blocks[0][0], gx_blocks[1][0],
                              gx_blocks[0][1], gx_blocks[1][1]], axis=0)
        gy = jnp.concatenate([gy_blocks[0][0], gy_blocks[1][0],
                              gy_blocks[0][1], gy_blocks[1][1]], axis=0)
        rois.append(dict(x1=x1, y1=y1, x2=x2, y2=y2, gx=gx, gy=gy,
                         lx3=lx3, vx3=vx3, ly3=ly3, vy3=vy3))

    gx_all = jnp.concatenate([r["gx"] for r in rois], axis=0)
    th = lax.dot_general(gx_all, hft_ref[0], dn, precision=_HI,
                         preferred_element_type=f32)
    tv = lax.dot_general(gx_all, vft_ref[0], dn, precision=_HI,
                         preferred_element_type=f32)

    def bilinear_mean(u, lx3, vx3, ly3, vy3):
        quad = [[None, None], [None, None]]
        for p in range(2):
            for q in range(2):
                v00 = u[5 * q:5 * q + 5, 5 * p:5 * p + 5, :]
                v01 = u[5 * q:5 * q + 5, 5 * (2 + p):5 * (2 + p) + 5, :]
                v10 = u[5 * (2 + q):5 * (2 + q) + 5, 5 * p:5 * p + 5, :]
                v11 = u[5 * (2 + q):5 * (2 + q) + 5,
                        5 * (2 + p):5 * (2 + p) + 5, :]
                o = (v00 * ((1 - ly3[q]) * (1 - lx3[p]))
                     + v01 * ((1 - ly3[q]) * lx3[p])
                     + v10 * (ly3[q] * (1 - lx3[p]))
                     + v11 * (ly3[q] * lx3[p]))
                quad[p][q] = jnp.where(vy3[q] & vx3[p], o, 0.0)
        return ((quad[0][0] + quad[0][1]) + (quad[1][0] + quad[1][1])) / 4

    for r in range(K):
        roi = rois[r]
        th_r = th[20 * r:20 * r + 20]
        tv_r = tv[20 * r:20 * r + 20]
        uh = lax.dot_general(roi["gy"], th_r, dn, precision=_HI,
                             preferred_element_type=f32)
        uv = lax.dot_general(gyv, tv_r, dn, precision=_HI,
                             preferred_element_type=f32)
        mh = bilinear_mean(uh, roi["lx3"], roi["vx3"], roi["ly3"], roi["vy3"])
        mv = bilinear_mean(uv, roi["lx3"], roi["vx3"], lyv3, vyv3)
        afh = lax.dot_general(mh.reshape(25, C), pool, (((0,), (0,)), ((), ())),
                              preferred_element_type=f32)
        afv = lax.dot_general(mv.reshape(25, C), pool, (((0,), (0,)), ((), ())),
                              preferred_element_type=f32)
        x1, y1, x2, y2 = roi["x1"], roi["y1"], roi["x2"], roi["y2"]
        v4h = jnp.concatenate([x1 / 64.0, y1 / 64.0, x2 / 64.0, y2 / 64.0],
                              axis=1)
        zero = x1 * 0.0
        v4v = jnp.concatenate([x1 / 64.0, zero, x2 / 64.0, zero + 1.0], axis=1)
        embh = jnp.transpose(
            lax.dot_general(v4h, ewt, (((1,), (0,)), ((), ())),
                            preferred_element_type=f32) + eb)
        embv = jnp.transpose(
            lax.dot_general(v4v, ewt, (((1,), (0,)), ((), ())),
                            preferred_element_type=f32) + eb)
        afh = afh + embh
        afv = afv + embv
        gh = lax.dot_general(pw, afh, (((1,), (0,)), ((), ())),
                             preferred_element_type=f32) + pbc
        gv = lax.dot_general(pw, afv, (((1,), (0,)), ((), ())),
                             preferred_element_type=f32) + pbc
        s = gh + gv
        wh = jnp.exp(gh) / s
        wv = jnp.exp(gv) / s
        o_ref[0, r] = gh * wh + gv * wv


@jax.jit
def kernel(hor_heatmap, hor_offset, hor_bsize, hor_kpt_feats, ver_kpt_feats,
           embed_w, embed_b, proj_w, proj_b):
    n = hor_heatmap.shape[0]
    hft = jnp.transpose(hor_kpt_feats, (0, 2, 3, 1))
    vft = jnp.transpose(ver_kpt_feats, (0, 2, 3, 1))
    ewt = jnp.transpose(embed_w)
    eb = embed_b.reshape(1, C)
    pbc = proj_b.reshape(C, 1)
    pool = jnp.asarray(_P)
    return pl.pallas_call(
        _body,
        grid=(n,),
        in_specs=[
            pl.BlockSpec((1, 1, HM, HM), lambda i: (i, 0, 0, 0)),
            pl.BlockSpec((1, 2, HM, HM), lambda i: (i, 0, 0, 0)),
            pl.BlockSpec((1, 4, HM, HM), lambda i: (i, 0, 0, 0)),
            pl.BlockSpec((1, HM, HM, C), lambda i: (i, 0, 0, 0)),
            pl.BlockSpec((1, HM, HM, C), lambda i: (i, 0, 0, 0)),
            pl.BlockSpec((4, C), lambda i: (0, 0)),
            pl.BlockSpec((1, C), lambda i: (0, 0)),
            pl.BlockSpec((C, C), lambda i: (0, 0)),
            pl.BlockSpec((C, 1), lambda i: (0, 0)),
            pl.BlockSpec((25, 14), lambda i: (0, 0)),
        ],
        out_specs=pl.BlockSpec((1, K, C, 14), lambda i: (i, 0, 0, 0)),
        out_shape=jax.ShapeDtypeStruct((n, K, C, 14), jnp.float32),
        compiler_params=pltpu.CompilerParams(
            dimension_semantics=("parallel",),
            vmem_limit_bytes=48 * 1024 * 1024,
        ),
    )(hor_heatmap, hor_offset, hor_bsize, hft, vft, ewt, eb, proj_w, pbc, pool)


# speedup vs baseline: 2.6209x; 1.0881x over previous
import jax
import jax.numpy as jnp
import numpy as np
from jax import lax
from jax.experimental import pallas as pl
from jax.experimental.pallas import tpu as pltpu

K = 4
HM = 64
OUT = 5
C = 128

_P = np.zeros((25, 14), np.float32)
for _i in range(14):
    _a = (_i * 25) // 14
    _b = int(np.ceil((_i + 1) * 25 / 14))
    _P[_a:_b, _i] = 1.0 / (_b - _a)

_HI = lax.Precision.HIGHEST

_VROWS = np.zeros(20, np.int64)
for _i in range(5):
    for _p in range(2):
        _s = np.float32(np.float32(_i + 0.25 + 0.5 * _p) * np.float32(64.0 / 5))
        _fl = int(np.floor(np.clip(_s, 0.0, 63.0)))
        _VROWS[4 * _i + 2 * _p] = _fl
        _VROWS[4 * _i + 2 * _p + 1] = min(_fl + 1, 63)


def _ax(lo, d, offs):
    s = lo + offs * d
    v = (s > -1.0) & (s < float(HM))
    sc = jnp.clip(s, 0.0, float(HM - 1))
    fl = jnp.floor(sc)
    fr = sc - fl
    fh = jnp.minimum(fl + 1.0, float(HM - 1))
    return v, fl, fh, fr


def _body(hm_ref, off_ref, bsz_ref, hft_ref, vft_ref, ewt_ref, eb_ref,
          pw_ref, pbc_ref, pool_ref, o_ref):
    f32 = jnp.float32
    dn = (((1,), (1,)), ((), ()))
    hm = hm_ref[0, 0]
    ri = lax.broadcasted_iota(jnp.int32, (HM, HM), 0)
    ci = lax.broadcasted_iota(jnp.int32, (HM, HM), 1)
    lin = ri * HM + ci
    pos64 = lax.broadcasted_iota(jnp.int32, (OUT, HM), 1).astype(f32)
    offs_c = [lax.broadcasted_iota(jnp.int32, (OUT, 1), 0).astype(f32) + 0.25,
              lax.broadcasted_iota(jnp.int32, (OUT, 1), 0).astype(f32) + 0.75]
    offs_x3 = [lax.broadcasted_iota(jnp.int32, (1, OUT, 1), 1).astype(f32) + 0.25,
               lax.broadcasted_iota(jnp.int32, (1, OUT, 1), 1).astype(f32) + 0.75]
    offs_y3 = [lax.broadcasted_iota(jnp.int32, (OUT, 1, 1), 0).astype(f32) + 0.25,
               lax.broadcasted_iota(jnp.int32, (OUT, 1, 1), 0).astype(f32) + 0.75]

    def onehot(coord):
        return (pos64 == coord).astype(f32)

    rhv5 = jnp.float32(64.0 / OUT)
    pos20 = lax.broadcasted_iota(jnp.int32, (OUT, 20), 1)
    bin20 = lax.broadcasted_iota(jnp.int32, (OUT, 1), 0) * 4
    gyv = jnp.concatenate(
        [(pos20 == bin20 + c).astype(f32) for c in (0, 2, 1, 3)], axis=0)
    lyv3, vyv3 = [], []
    for par in range(2):
        vv3, _, _, frv3 = _ax(0.0, rhv5, offs_y3[par])
        lyv3.append(frv3)
        vyv3.append(vv3)

    ewt = ewt_ref[...]
    eb = eb_ref[...]
    pw = pw_ref[...]
    pbc = pbc_ref[...]
    pool = pool_ref[...]

    cur = hm
    rois = []
    for _ in range(K):
        mx = jnp.max(cur, axis=(0, 1), keepdims=True)
        idx = jnp.min(jnp.where(cur == mx, lin, HM * HM),
                      axis=(0, 1), keepdims=True)
        sel = lin == idx
        cur = jnp.where(sel, -jnp.inf, cur)
        self_ = sel.astype(f32)
        y = (idx // HM).astype(f32)
        x = (idx % HM).astype(f32)
        o0 = jnp.sum(off_ref[0, 0] * self_, axis=(0, 1), keepdims=True)
        o1 = jnp.sum(off_ref[0, 1] * self_, axis=(0, 1), keepdims=True)
        b0 = jnp.sum(bsz_ref[0, 0] * self_, axis=(0, 1), keepdims=True)
        b1 = jnp.sum(bsz_ref[0, 1] * self_, axis=(0, 1), keepdims=True)
        b2 = jnp.sum(bsz_ref[0, 2] * self_, axis=(0, 1), keepdims=True)
        b3 = jnp.sum(bsz_ref[0, 3] * self_, axis=(0, 1), keepdims=True)
        wdt = b0 + b2
        hgt = b1 + b3
        cx = (x + o0) * 4.0
        cy = (y + o1) * 4.0
        x1 = (cx - wdt / 2) * 0.25
        y1 = (cy - hgt / 2) * 0.25
        x2 = (cx + wdt / 2) * 0.25
        y2 = (cy + hgt / 2) * 0.25
        rw5 = jnp.maximum(x2 - x1, 1.0) / OUT
        rh5 = jnp.maximum(y2 - y1, 1.0) / OUT
        gx_blocks, gy_blocks = [], []
        lx3, vx3, ly3, vy3 = [], [], [], []
        for par in range(2):
            _, flx, fhx, _ = _ax(x1, rw5, offs_c[par])
            gx_blocks.append((onehot(flx), onehot(fhx)))
            _, fly, fhy, _ = _ax(y1, rh5, offs_c[par])
            gy_blocks.append((onehot(fly), onehot(fhy)))
            vx, _, _, frx = _ax(x1, rw5, offs_x3[par])
            lx3.append(frx)
            vx3.append(vx)
            vy, _, _, fry = _ax(y1, rh5, offs_y3[par])
            ly3.append(fry)
            vy3.append(vy)
        gx = jnp.concatenate([gx_blocks[0][0], gx_blocks[1][0],
                              gx_blocks[0][1], gx_blocks[1][1]], axis=0)
        gy = jnp.concatenate([gy_blocks[0][0], gy_blocks[1][0],
                              gy_blocks[0][1], gy_blocks[1][1]], axis=0)
        rois.append(dict(x1=x1, y1=y1, x2=x2, y2=y2, gx=gx, gy=gy,
                         lx3=lx3, vx3=vx3, ly3=ly3, vy3=vy3))

    gx_all = jnp.concatenate([r["gx"] for r in rois], axis=0)
    th = lax.dot_general(gx_all, hft_ref[0], dn, precision=_HI,
                         preferred_element_type=f32)
    tv = lax.dot_general(gx_all, vft_ref[0], dn, precision=_HI,
                         preferred_element_type=f32)

    def bilinear_mean(u, lx3, vx3, ly3, vy3):
        quad = [[None, None], [None, None]]
        for p in range(2):
            for q in range(2):
                v00 = u[5 * q:5 * q + 5, 5 * p:5 * p + 5, :]
                v01 = u[5 * q:5 * q + 5, 5 * (2 + p):5 * (2 + p) + 5, :]
                v10 = u[5 * (2 + q):5 * (2 + q) + 5, 5 * p:5 * p + 5, :]
                v11 = u[5 * (2 + q):5 * (2 + q) + 5,
                        5 * (2 + p):5 * (2 + p) + 5, :]
                o = (v00 * ((1 - ly3[q]) * (1 - lx3[p]))
                     + v01 * ((1 - ly3[q]) * lx3[p])
                     + v10 * (ly3[q] * (1 - lx3[p]))
                     + v11 * (ly3[q] * lx3[p]))
                quad[p][q] = jnp.where(vy3[q] & vx3[p], o, 0.0)
        return ((quad[0][0] + quad[0][1]) + (quad[1][0] + quad[1][1])) / 4

    for r in range(K):
        roi = rois[r]
        th_r = th[20 * r:20 * r + 20]
        tv_r = tv[20 * r:20 * r + 20]
        uh = lax.dot_general(roi["gy"], th_r, dn, precision=_HI,
                             preferred_element_type=f32)
        uv = lax.dot_general(gyv, tv_r, dn, precision=_HI,
                             preferred_element_type=f32)
        mh = bilinear_mean(uh, roi["lx3"], roi["vx3"], roi["ly3"], roi["vy3"])
        mv = bilinear_mean(uv, roi["lx3"], roi["vx3"], lyv3, vyv3)
        afh = lax.dot_general(mh.reshape(25, C), pool, (((0,), (0,)), ((), ())),
                              preferred_element_type=f32)
        afv = lax.dot_general(mv.reshape(25, C), pool, (((0,), (0,)), ((), ())),
                              preferred_element_type=f32)
        x1, y1, x2, y2 = roi["x1"], roi["y1"], roi["x2"], roi["y2"]
        v4h = jnp.concatenate([x1 / 64.0, y1 / 64.0, x2 / 64.0, y2 / 64.0],
                              axis=1)
        zero = x1 * 0.0
        v4v = jnp.concatenate([x1 / 64.0, zero, x2 / 64.0, zero + 1.0], axis=1)
        embh = jnp.transpose(
            lax.dot_general(v4h, ewt, (((1,), (0,)), ((), ())),
                            preferred_element_type=f32) + eb)
        embv = jnp.transpose(
            lax.dot_general(v4v, ewt, (((1,), (0,)), ((), ())),
                            preferred_element_type=f32) + eb)
        afh = afh + embh
        afv = afv + embv
        gh = lax.dot_general(pw, afh, (((1,), (0,)), ((), ())),
                             preferred_element_type=f32) + pbc
        gv = lax.dot_general(pw, afv, (((1,), (0,)), ((), ())),
                             preferred_element_type=f32) + pbc
        s = gh + gv
        wh = jnp.exp(gh) / s
        wv = jnp.exp(gv) / s
        o_ref[0, r] = gh * wh + gv * wv


@jax.jit
def kernel(hor_heatmap, hor_offset, hor_bsize, hor_kpt_feats, ver_kpt_feats,
           embed_w, embed_b, proj_w, proj_b):
    n = hor_heatmap.shape[0]
    hft = jnp.transpose(hor_kpt_feats, (0, 2, 3, 1))
    vft = jnp.transpose(ver_kpt_feats[:, :, _VROWS, :], (0, 2, 3, 1))
    ewt = jnp.transpose(embed_w)
    eb = embed_b.reshape(1, C)
    pbc = proj_b.reshape(C, 1)
    pool = jnp.asarray(_P)
    return pl.pallas_call(
        _body,
        grid=(n,),
        in_specs=[
            pl.BlockSpec((1, 1, HM, HM), lambda i: (i, 0, 0, 0)),
            pl.BlockSpec((1, 2, HM, HM), lambda i: (i, 0, 0, 0)),
            pl.BlockSpec((1, 4, HM, HM), lambda i: (i, 0, 0, 0)),
            pl.BlockSpec((1, HM, HM, C), lambda i: (i, 0, 0, 0)),
            pl.BlockSpec((1, 20, HM, C), lambda i: (i, 0, 0, 0)),
            pl.BlockSpec((4, C), lambda i: (0, 0)),
            pl.BlockSpec((1, C), lambda i: (0, 0)),
            pl.BlockSpec((C, C), lambda i: (0, 0)),
            pl.BlockSpec((C, 1), lambda i: (0, 0)),
            pl.BlockSpec((25, 14), lambda i: (0, 0)),
        ],
        out_specs=pl.BlockSpec((1, K, C, 14), lambda i: (i, 0, 0, 0)),
        out_shape=jax.ShapeDtypeStruct((n, K, C, 14), jnp.float32),
        compiler_params=pltpu.CompilerParams(
            dimension_semantics=("parallel",),
            vmem_limit_bytes=48 * 1024 * 1024,
        ),
    )(hor_heatmap, hor_offset, hor_bsize, hft, vft, ewt, eb, proj_w, pbc, pool)
